# baseline (speedup 1.0000x reference)
import jax
import jax.numpy as jnp
from jax.experimental import pallas as pl

N = 10000
E = 160000
D = 256
H = 8
C = 32
HC = H * C
NUM_CLASSES = 64
BN = 1000


def _mm_kernel(x_ref, w_ref, o_ref):
    o_ref[...] = jnp.dot(x_ref[...], w_ref[...], preferred_element_type=jnp.float32)


def _matmul(x, w):
    n, k = x.shape
    _, m = w.shape
    return pl.pallas_call(
        _mm_kernel,
        grid=(n // BN,),
        in_specs=[
            pl.BlockSpec((BN, k), lambda i: (i, 0)),
            pl.BlockSpec((k, m), lambda i: (0, 0)),
        ],
        out_specs=pl.BlockSpec((BN, m), lambda i: (i, 0)),
        out_shape=jax.ShapeDtypeStruct((n, m), jnp.float32),
    )(x, w)


def _gat(x, src, dst, W, a_src, a_dst, b, heads, c_out, concat):
    h = _matmul(x, W).reshape(N, heads, c_out)
    alpha_s = (h * a_src[None]).sum(-1)
    alpha_d = (h * a_dst[None]).sum(-1)
    e = alpha_s[src] + alpha_d[dst]
    e = jax.nn.leaky_relu(e, 0.2)
    e_max = jax.ops.segment_max(e, dst, num_segments=N)
    e_exp = jnp.exp(e - e_max[dst])
    denom = jax.ops.segment_sum(e_exp, dst, num_segments=N)
    alpha = e_exp / (denom[dst] + 1e-16)
    msg = h[src] * alpha[:, :, None]
    out = jax.ops.segment_sum(msg, dst, num_segments=N)
    if concat:
        out = out.reshape(N, heads * c_out)
    else:
        out = out.mean(axis=1)
    return out + b


def _ln(x, g, b, eps=1e-5):
    mu = x.mean(-1, keepdims=True)
    var = ((x - mu) ** 2).mean(-1, keepdims=True)
    return (x - mu) / jnp.sqrt(var + eps) * g + b


def kernel(x, edge_index, W1, a_src1, a_dst1, b1, ln1_g, ln1_b, W2, a_src2, a_dst2, b2, ln2_g, ln2_b, W3, a_src3, a_dst3, b3):
    loop = jnp.arange(N, dtype=edge_index.dtype)
    src = jnp.concatenate([edge_index[0], loop])
    dst = jnp.concatenate([edge_index[1], loop])
    h = _gat(x, src, dst, W1, a_src1, a_dst1, b1, H, C, True)
    h = _ln(h + x, ln1_g, ln1_b)
    h = jax.nn.elu(h)
    h2 = _gat(h, src, dst, W2, a_src2, a_dst2, b2, H, C, True)
    h2 = _ln(h2 + h, ln2_g, ln2_b)
    h2 = jax.nn.elu(h2)
    return _gat(h2, src, dst, W3, a_src3, a_dst3, b3, 1, NUM_CLASSES, False)


# speedup vs baseline: 16.7773x; 16.7773x over previous
import functools

import jax
import jax.numpy as jnp
from jax import lax
from jax.experimental import pallas as pl
from jax.experimental.pallas import tpu as pltpu
from jax.experimental.pallas import tpu_sc as plsc

N = 10000
E = 160000
D = 256
H = 8
C = 32
HC = H * C
NUM_CLASSES = 64

NSUB = 16
NP = 10112
TRASH = N
E2 = E + N
B = 128
E2P = 172032
CHUNK = E2P // NSUB
NBATCH = CHUNK // B
ROWS = NP // NSUB

BN = 1000


def _tc_first_body(x_ref, w_ref, as_ref, ad_ref, h0_ref, h1_ref, so_ref, do_ref):
    h = jnp.dot(x_ref[...], w_ref[...], preferred_element_type=jnp.float32)
    half = h.shape[1] // 2
    h0_ref[...] = h[:, :half]
    h1_ref[...] = h[:, half:]
    so_ref[...] = jnp.dot(h, as_ref[...], preferred_element_type=jnp.float32)
    do_ref[...] = jnp.dot(h, ad_ref[...], preferred_element_type=jnp.float32)


def _tc_mid_body(z_ref, r_ref, g_ref, b_ref, w_ref, as_ref, ad_ref,
                 y_ref, h0_ref, h1_ref, so_ref, do_ref):
    z = z_ref[...] + r_ref[...]
    mu = jnp.mean(z, axis=-1, keepdims=True)
    var = jnp.mean((z - mu) ** 2, axis=-1, keepdims=True)
    y = (z - mu) / jnp.sqrt(var + 1e-5) * g_ref[...] + b_ref[...]
    y = jnp.where(y > 0, y, jnp.exp(jnp.minimum(y, 0.0)) - 1.0)
    y_ref[...] = y
    h = jnp.dot(y, w_ref[...], preferred_element_type=jnp.float32)
    half = h.shape[1] // 2
    h0_ref[...] = h[:, :half]
    h1_ref[...] = h[:, half:]
    so_ref[...] = jnp.dot(h, as_ref[...], preferred_element_type=jnp.float32)
    do_ref[...] = jnp.dot(h, ad_ref[...], preferred_element_type=jnp.float32)


def _tc_first(x, w, a_s, a_d):
    k, m = w.shape
    half = m // 2
    f32 = jnp.float32
    return pl.pallas_call(
        _tc_first_body,
        grid=(N // BN,),
        in_specs=[
            pl.BlockSpec((BN, k), lambda i: (i, 0)),
            pl.BlockSpec((k, m), lambda i: (0, 0)),
            pl.BlockSpec((m, H), lambda i: (0, 0)),
            pl.BlockSpec((m, H), lambda i: (0, 0)),
        ],
        out_specs=[
            pl.BlockSpec((BN, half), lambda i: (i, 0)),
            pl.BlockSpec((BN, half), lambda i: (i, 0)),
            pl.BlockSpec((BN, H), lambda i: (i, 0)),
            pl.BlockSpec((BN, H), lambda i: (i, 0)),
        ],
        out_shape=[
            jax.ShapeDtypeStruct((N, half), f32),
            jax.ShapeDtypeStruct((N, half), f32),
            jax.ShapeDtypeStruct((N, H), f32),
            jax.ShapeDtypeStruct((N, H), f32),
        ],
    )(x, w, a_s, a_d)


def _tc_mid(z, r, ln_g, ln_b, w, a_s, a_d):
    k, m = w.shape
    half = m // 2
    f32 = jnp.float32
    return pl.pallas_call(
        _tc_mid_body,
        grid=(N // BN,),
        in_specs=[
            pl.BlockSpec((BN, k), lambda i: (i, 0)),
            pl.BlockSpec((BN, k), lambda i: (i, 0)),
            pl.BlockSpec((1, k), lambda i: (0, 0)),
            pl.BlockSpec((1, k), lambda i: (0, 0)),
            pl.BlockSpec((k, m), lambda i: (0, 0)),
            pl.BlockSpec((m, H), lambda i: (0, 0)),
            pl.BlockSpec((m, H), lambda i: (0, 0)),
        ],
        out_specs=[
            pl.BlockSpec((BN, k), lambda i: (i, 0)),
            pl.BlockSpec((BN, half), lambda i: (i, 0)),
            pl.BlockSpec((BN, half), lambda i: (i, 0)),
            pl.BlockSpec((BN, H), lambda i: (i, 0)),
            pl.BlockSpec((BN, H), lambda i: (i, 0)),
        ],
        out_shape=[
            jax.ShapeDtypeStruct((N, k), f32),
            jax.ShapeDtypeStruct((N, half), f32),
            jax.ShapeDtypeStruct((N, half), f32),
            jax.ShapeDtypeStruct((N, H), f32),
            jax.ShapeDtypeStruct((N, H), f32),
        ],
    )(z, r, ln_g.reshape(1, k), ln_b.reshape(1, k), w, a_s, a_d)


def _splat(v, idx):
    dnums = lax.GatherDimensionNumbers(
        offset_dims=(), collapsed_slice_dims=(0,), start_index_map=(0,))
    return lax.gather(v, idx[:, None], dnums, (1,),
                      mode=lax.GatherScatterMode.PROMISE_IN_BOUNDS)


@functools.cache
def _make_sc_layer(half, c_l, interpret=False):
    nj = half // 16
    f32 = jnp.float32
    mesh = plsc.VectorSubcoreMesh(core_axis_name="c", subcore_axis_name="s",
                                  num_cores=2, num_subcores=NSUB)

    def body(src_h, dst_h, ast_h, adt_h, h0_h, h1_h, bias_h, o0_h, o1_h,
             denom, acc, sbuf, dbuf, asb, adb, pbuf, dnb, g, bias_v, sem):
        c = lax.axis_index("c")
        s = lax.axis_index("s")
        ebase = s * CHUNK
        rbase = s * ROWS

        pltpu.sync_copy(bias_h.at[pl.ds(c, 1)], bias_v)

        def _fill(i, _):
            for j in range(nj):
                g[i, pl.ds(16 * j, 16)] = bias_v[0, pl.ds(16 * j, 16)]
            pbuf[i] = jnp.zeros((16,), f32)
            return 0
        lax.fori_loop(0, B, _fill, 0)
        for k in range(4):
            pltpu.sync_copy(g.at[pl.ds(0, B)], acc.at[pl.ds(rbase + k * B, B)])
            pltpu.sync_copy(pbuf.at[pl.ds(0, B)], denom.at[pl.ds(rbase + k * B, B)])
        pltpu.sync_copy(g.at[pl.ds(0, ROWS - 4 * B)], acc.at[pl.ds(rbase + 4 * B, ROWS - 4 * B)])
        pltpu.sync_copy(pbuf.at[pl.ds(0, ROWS - 4 * B)], denom.at[pl.ds(rbase + 4 * B, ROWS - 4 * B)])
        plsc.subcore_barrier()

        def p1(it, _):
            off = ebase + it * B
            pltpu.sync_copy(src_h.at[pl.ds(off, B)], sbuf)
            pltpu.sync_copy(dst_h.at[pl.ds(off, B)], dbuf)
            pltpu.async_copy(ast_h.at[sbuf], asb, sem).wait()
            pltpu.async_copy(adt_h.at[dbuf], adb, sem).wait()

            def pe(i, _):
                e = asb[i] + adb[i]
                e = jnp.maximum(e, 0.2 * e)
                pbuf[i] = jnp.exp(e)
                return 0
            lax.fori_loop(0, B, pe, 0)
            pltpu.sync_copy(pbuf, denom.at[dbuf], add=True)
            return 0
        lax.fori_loop(0, NBATCH, p1, 0)
        plsc.subcore_barrier()

        def p2(it, _):
            off = ebase + it * B
            pltpu.sync_copy(src_h.at[pl.ds(off, B)], sbuf)
            pltpu.sync_copy(dst_h.at[pl.ds(off, B)], dbuf)
            pltpu.async_copy(ast_h.at[sbuf], asb, sem).wait()
            pltpu.async_copy(adt_h.at[dbuf], adb, sem).wait()
            pltpu.async_copy(denom.at[dbuf], dnb, sem).wait()

            @pl.when(c == 0)
            def _():
                pltpu.async_copy(h0_h.at[sbuf], g, sem).wait()

            @pl.when(c == 1)
            def _():
                pltpu.async_copy(h1_h.at[sbuf], g, sem).wait()

            def me(i, _):
                e = asb[i] + adb[i]
                e = jnp.maximum(e, 0.2 * e)
                a = jnp.exp(e) / (dnb[i] + 1e-16)
                if c_l == 32:
                    cbase = c * (half // 32)
                    for j in range(nj):
                        idx = jnp.full((16,), j // 2, jnp.int32) + cbase
                        sp = _splat(a, idx)
                        g[i, pl.ds(16 * j, 16)] = g[i, pl.ds(16 * j, 16)] * sp
                else:
                    sp = _splat(a, jnp.zeros((16,), jnp.int32))
                    for j in range(nj):
                        g[i, pl.ds(16 * j, 16)] = g[i, pl.ds(16 * j, 16)] * sp
                return 0
            lax.fori_loop(0, B, me, 0)
            pltpu.sync_copy(g, acc.at[dbuf], add=True)
            return 0
        lax.fori_loop(0, NBATCH, p2, 0)
        plsc.subcore_barrier()

        @pl.when(c == 0)
        def _():
            pltpu.sync_copy(acc.at[pl.ds(rbase, ROWS)], o0_h.at[pl.ds(rbase, ROWS)])

        @pl.when(c == 1)
        def _():
            pltpu.sync_copy(acc.at[pl.ds(rbase, ROWS)], o1_h.at[pl.ds(rbase, ROWS)])

    return pl.kernel(
        body,
        out_type=(
            jax.ShapeDtypeStruct((NP, half), f32),
            jax.ShapeDtypeStruct((NP, half), f32),
        ),
        mesh=mesh,
        interpret=interpret,
        compiler_params=pltpu.CompilerParams(use_tc_tiling_on_sc=False),
        scratch_types=[
            pltpu.VMEM_SHARED((NP, 16), f32),
            pltpu.VMEM_SHARED((NP, half), f32),
            pltpu.VMEM((B,), jnp.int32),
            pltpu.VMEM((B,), jnp.int32),
            pltpu.VMEM((B, 16), f32),
            pltpu.VMEM((B, 16), f32),
            pltpu.VMEM((B, 16), f32),
            pltpu.VMEM((B, 16), f32),
            pltpu.VMEM((B, half), f32),
            pltpu.VMEM((1, half), f32),
            pltpu.SemaphoreType.DMA,
        ],
    )


def _attn_mats(a_src, a_dst, heads, c_out):
    eye = jnp.eye(heads, dtype=jnp.float32)
    a_s = (a_src[:, :, None] * eye[:, None, :]).reshape(heads * c_out, heads)
    a_d = (a_dst[:, :, None] * eye[:, None, :]).reshape(heads * c_out, heads)
    if heads < H:
        a_s = jnp.pad(a_s, ((0, 0), (0, H - heads)))
        a_d = jnp.pad(a_d, ((0, 0), (0, H - heads)))
    return a_s, a_d


def _pad_tab(a):
    return jnp.pad(a, ((0, NP - N), (0, 16 - H)))


def _sc_edge_stage(fn, src, dst, alo, ahi, h0, h1, bias):
    ast = _pad_tab(alo)
    adt = _pad_tab(ahi)
    half = h0.shape[1]
    bias2 = bias.reshape(2, half)
    o0, o1 = fn(src, dst, ast, adt, h0, h1, bias2)
    return jnp.concatenate([o0[:N], o1[:N]], axis=1)


def kernel(x, edge_index, W1, a_src1, a_dst1, b1, ln1_g, ln1_b,
           W2, a_src2, a_dst2, b2, ln2_g, ln2_b, W3, a_src3, a_dst3, b3):
    loop = jnp.arange(N, dtype=edge_index.dtype)
    pad = E2P - E2
    src = jnp.concatenate([edge_index[0], loop, jnp.zeros((pad,), edge_index.dtype)])
    dst = jnp.concatenate([edge_index[1], loop, jnp.full((pad,), TRASH, edge_index.dtype)])

    sc_full = _make_sc_layer(HC // 2, C)
    sc_out = _make_sc_layer(NUM_CLASSES // 2, 64)

    as1, ad1 = _attn_mats(a_src1, a_dst1, H, C)
    h0, h1, als, ald = _tc_first(x, W1, as1, ad1)
    acc1 = _sc_edge_stage(sc_full, src, dst, als, ald, h0, h1, b1)

    as2, ad2 = _attn_mats(a_src2, a_dst2, H, C)
    y1, h0, h1, als, ald = _tc_mid(acc1, x, ln1_g, ln1_b, W2, as2, ad2)
    acc2 = _sc_edge_stage(sc_full, src, dst, als, ald, h0, h1, b2)

    as3, ad3 = _attn_mats(a_src3, a_dst3, 1, NUM_CLASSES)
    _, h0, h1, als, ald = _tc_mid(acc2, y1, ln2_g, ln2_b, W3, as3, ad3)
    return _sc_edge_stage(sc_out, src, dst, als, ald, h0, h1, b3)


# speedup vs baseline: 32.8932x; 1.9606x over previous
import functools

import jax
import jax.numpy as jnp
from jax import lax
from jax.experimental import pallas as pl
from jax.experimental.pallas import tpu as pltpu
from jax.experimental.pallas import tpu_sc as plsc

N = 10000
E = 160000
D = 256
H = 8
C = 32
HC = H * C
NUM_CLASSES = 64

NSUB = 16
NP = 10112
TRASH = N
E2 = E + N
B = 128
E2P = 172032
CHUNK = E2P // NSUB
NBATCH = CHUNK // B
ROWS = NP // NSUB

BN = 1000


def _tc_first_body(x_ref, w_ref, as_ref, ad_ref, h0_ref, h1_ref, so_ref, do_ref):
    h = jnp.dot(x_ref[...], w_ref[...], preferred_element_type=jnp.float32)
    half = h.shape[1] // 2
    h0_ref[...] = h[:, :half]
    h1_ref[...] = h[:, half:]
    so_ref[...] = jnp.dot(h, as_ref[...], preferred_element_type=jnp.float32)
    do_ref[...] = jnp.dot(h, ad_ref[...], preferred_element_type=jnp.float32)


def _tc_mid_body(z_ref, r_ref, g_ref, b_ref, w_ref, as_ref, ad_ref,
                 y_ref, h0_ref, h1_ref, so_ref, do_ref):
    z = z_ref[...] + r_ref[...]
    mu = jnp.mean(z, axis=-1, keepdims=True)
    var = jnp.mean((z - mu) ** 2, axis=-1, keepdims=True)
    y = (z - mu) / jnp.sqrt(var + 1e-5) * g_ref[...] + b_ref[...]
    y = jnp.where(y > 0, y, jnp.exp(jnp.minimum(y, 0.0)) - 1.0)
    y_ref[...] = y
    h = jnp.dot(y, w_ref[...], preferred_element_type=jnp.float32)
    half = h.shape[1] // 2
    h0_ref[...] = h[:, :half]
    h1_ref[...] = h[:, half:]
    so_ref[...] = jnp.dot(h, as_ref[...], preferred_element_type=jnp.float32)
    do_ref[...] = jnp.dot(h, ad_ref[...], preferred_element_type=jnp.float32)


def _tc_first(x, w, a_s, a_d):
    k, m = w.shape
    half = m // 2
    f32 = jnp.float32
    return pl.pallas_call(
        _tc_first_body,
        grid=(N // BN,),
        in_specs=[
            pl.BlockSpec((BN, k), lambda i: (i, 0)),
            pl.BlockSpec((k, m), lambda i: (0, 0)),
            pl.BlockSpec((m, H), lambda i: (0, 0)),
            pl.BlockSpec((m, H), lambda i: (0, 0)),
        ],
        out_specs=[
            pl.BlockSpec((BN, half), lambda i: (i, 0)),
            pl.BlockSpec((BN, half), lambda i: (i, 0)),
            pl.BlockSpec((BN, H), lambda i: (i, 0)),
            pl.BlockSpec((BN, H), lambda i: (i, 0)),
        ],
        out_shape=[
            jax.ShapeDtypeStruct((N, half), f32),
            jax.ShapeDtypeStruct((N, half), f32),
            jax.ShapeDtypeStruct((N, H), f32),
            jax.ShapeDtypeStruct((N, H), f32),
        ],
    )(x, w, a_s, a_d)


def _tc_mid(z, r, ln_g, ln_b, w, a_s, a_d):
    k, m = w.shape
    half = m // 2
    f32 = jnp.float32
    return pl.pallas_call(
        _tc_mid_body,
        grid=(N // BN,),
        in_specs=[
            pl.BlockSpec((BN, k), lambda i: (i, 0)),
            pl.BlockSpec((BN, k), lambda i: (i, 0)),
            pl.BlockSpec((1, k), lambda i: (0, 0)),
            pl.BlockSpec((1, k), lambda i: (0, 0)),
            pl.BlockSpec((k, m), lambda i: (0, 0)),
            pl.BlockSpec((m, H), lambda i: (0, 0)),
            pl.BlockSpec((m, H), lambda i: (0, 0)),
        ],
        out_specs=[
            pl.BlockSpec((BN, k), lambda i: (i, 0)),
            pl.BlockSpec((BN, half), lambda i: (i, 0)),
            pl.BlockSpec((BN, half), lambda i: (i, 0)),
            pl.BlockSpec((BN, H), lambda i: (i, 0)),
            pl.BlockSpec((BN, H), lambda i: (i, 0)),
        ],
        out_shape=[
            jax.ShapeDtypeStruct((N, k), f32),
            jax.ShapeDtypeStruct((N, half), f32),
            jax.ShapeDtypeStruct((N, half), f32),
            jax.ShapeDtypeStruct((N, H), f32),
            jax.ShapeDtypeStruct((N, H), f32),
        ],
    )(z, r, ln_g.reshape(1, k), ln_b.reshape(1, k), w, a_s, a_d)


def _splat(v, idx):
    dnums = lax.GatherDimensionNumbers(
        offset_dims=(), collapsed_slice_dims=(0,), start_index_map=(0,))
    return lax.gather(v, idx[:, None], dnums, (1,),
                      mode=lax.GatherScatterMode.PROMISE_IN_BOUNDS)


@functools.cache
def _make_sc_layer(half, c_l, interpret=False):
    nj = half // 16
    f32 = jnp.float32
    mesh = plsc.VectorSubcoreMesh(core_axis_name="c", subcore_axis_name="s",
                                  num_cores=2, num_subcores=NSUB)

    def _head_splat(p, c, j):
        if c_l == 32:
            idx = jnp.full((16,), j // 2, jnp.int32) + c * (half // 32)
        else:
            idx = jnp.zeros((16,), jnp.int32)
        return _splat(p, idx)

    def body(s2_h, d2_h, ast_h, adt_h, hcat_h, bias_h, o0_h, o1_h,
             denom, acc, sbuf, dbuf, sbuf2,
             asb, adb, pbuf, dnb, g, bias_v,
             sem_s, sem_d, sem_a, sem_b, sem_g, ssem_d, ssem_a):
        c = lax.axis_index("c")
        s = lax.axis_index("s")
        rbase = s * ROWS
        shift = c * NP

        pltpu.sync_copy(bias_h.at[pl.ds(c, 1)], bias_v)

        def _fill(i, _):
            for j in range(nj):
                g[i, pl.ds(16 * j, 16)] = jnp.zeros((16,), f32)
            pbuf[i] = jnp.zeros((16,), f32)
            return 0
        lax.fori_loop(0, B, _fill, 0)
        for k in range(4):
            pltpu.sync_copy(g.at[pl.ds(0, B)], acc.at[pl.ds(rbase + k * B, B)])
            pltpu.sync_copy(pbuf.at[pl.ds(0, B)], denom.at[pl.ds(rbase + k * B, B)])
        pltpu.sync_copy(g.at[pl.ds(0, ROWS - 4 * B)], acc.at[pl.ds(rbase + 4 * B, ROWS - 4 * B)])
        pltpu.sync_copy(pbuf.at[pl.ds(0, ROWS - 4 * B)], denom.at[pl.ds(rbase + 4 * B, ROWS - 4 * B)])
        plsc.subcore_barrier()

        def bstep(it, _):
            row = s * NBATCH + it
            cs = pltpu.async_copy(s2_h.at[pl.ds(row, 1)], sbuf, sem_s)
            cd = pltpu.async_copy(d2_h.at[pl.ds(row, 1)], dbuf, sem_d)

            @pl.when(it > 0)
            def _():
                pltpu.make_async_copy(pbuf, denom.at[dbuf.at[0]], ssem_d).wait()
                pltpu.make_async_copy(g, acc.at[dbuf.at[0]], ssem_a).wait()
            cs.wait()
            cd.wait()

            def sh(k, _):
                sbuf2[0, pl.ds(16 * k, 16)] = sbuf[0, pl.ds(16 * k, 16)] + shift
                return 0
            lax.fori_loop(0, B // 16, sh, 0)
            ca = pltpu.async_copy(ast_h.at[sbuf.at[0]], asb, sem_a)
            cb = pltpu.async_copy(adt_h.at[dbuf.at[0]], adb, sem_b)
            cg = pltpu.async_copy(hcat_h.at[sbuf2.at[0]], g, sem_g)
            ca.wait()
            cb.wait()

            def pe(i, _):
                e = asb[i] + adb[i]
                e = jnp.maximum(e, 0.2 * e)
                pbuf[i] = jnp.exp(e)
                return 0
            lax.fori_loop(0, B, pe, 0)
            cg.wait()

            def me(i, _):
                p = pbuf[i]
                for j in range(nj):
                    g[i, pl.ds(16 * j, 16)] = g[i, pl.ds(16 * j, 16)] * _head_splat(p, c, j)
                return 0
            lax.fori_loop(0, B, me, 0)
            pltpu.async_copy(pbuf, denom.at[dbuf.at[0]], ssem_d, add=True)
            pltpu.async_copy(g, acc.at[dbuf.at[0]], ssem_a, add=True)
            return 0
        lax.fori_loop(0, NBATCH, bstep, 0)
        pltpu.make_async_copy(pbuf, denom.at[dbuf.at[0]], ssem_d).wait()
        pltpu.make_async_copy(g, acc.at[dbuf.at[0]], ssem_a).wait()
        plsc.subcore_barrier()

        def norm_chunk(r0, nrows):
            pltpu.sync_copy(acc.at[pl.ds(r0, nrows)], g.at[pl.ds(0, nrows)])
            pltpu.sync_copy(denom.at[pl.ds(r0, nrows)], dnb.at[pl.ds(0, nrows)])

            def ne(i, _):
                d = dnb[i] + 1e-16
                for j in range(nj):
                    bv = bias_v[0, pl.ds(16 * j, 16)]
                    g[i, pl.ds(16 * j, 16)] = g[i, pl.ds(16 * j, 16)] / _head_splat(d, c, j) + bv
                return 0
            lax.fori_loop(0, nrows, ne, 0)

            @pl.when(c == 0)
            def _():
                pltpu.sync_copy(g.at[pl.ds(0, nrows)], o0_h.at[pl.ds(r0, nrows)])

            @pl.when(c == 1)
            def _():
                pltpu.sync_copy(g.at[pl.ds(0, nrows)], o1_h.at[pl.ds(r0, nrows)])

        for k in range(4):
            norm_chunk(rbase + k * B, B)
        norm_chunk(rbase + 4 * B, ROWS - 4 * B)

    return pl.kernel(
        body,
        out_type=(
            jax.ShapeDtypeStruct((NP, half), f32),
            jax.ShapeDtypeStruct((NP, half), f32),
        ),
        mesh=mesh,
        interpret=interpret,
        compiler_params=pltpu.CompilerParams(use_tc_tiling_on_sc=False),
        scratch_types=[
            pltpu.VMEM_SHARED((NP, 16), f32),
            pltpu.VMEM_SHARED((NP, half), f32),
            pltpu.VMEM((1, B), jnp.int32),
            pltpu.VMEM((1, B), jnp.int32),
            pltpu.VMEM((1, B), jnp.int32),
            pltpu.VMEM((B, 16), f32),
            pltpu.VMEM((B, 16), f32),
            pltpu.VMEM((B, 16), f32),
            pltpu.VMEM((B, 16), f32),
            pltpu.VMEM((B, half), f32),
            pltpu.VMEM((1, half), f32),
            pltpu.SemaphoreType.DMA,
            pltpu.SemaphoreType.DMA,
            pltpu.SemaphoreType.DMA,
            pltpu.SemaphoreType.DMA,
            pltpu.SemaphoreType.DMA,
            pltpu.SemaphoreType.DMA,
            pltpu.SemaphoreType.DMA,
        ],
    )


def _attn_mats(a_src, a_dst, heads, c_out):
    eye = jnp.eye(heads, dtype=jnp.float32)
    a_s = (a_src[:, :, None] * eye[:, None, :]).reshape(heads * c_out, heads)
    a_d = (a_dst[:, :, None] * eye[:, None, :]).reshape(heads * c_out, heads)
    if heads < H:
        a_s = jnp.pad(a_s, ((0, 0), (0, H - heads)))
        a_d = jnp.pad(a_d, ((0, 0), (0, H - heads)))
    return a_s, a_d


def _pad_tab(a):
    return jnp.pad(a, ((0, NP - N), (0, 16 - H)))


def _sc_edge_stage(fn, src2, dst2, alo, ahi, h0, h1, bias):
    ast = _pad_tab(alo)
    adt = _pad_tab(ahi)
    ast2 = jnp.concatenate([ast, ast], axis=0)
    adt2 = jnp.concatenate([adt, adt], axis=0)
    half = h0.shape[1]
    hcat = jnp.concatenate([jnp.pad(h0, ((0, NP - N), (0, 0))),
                            jnp.pad(h1, ((0, NP - N), (0, 0)))], axis=0)
    bias2 = bias.reshape(2, half)
    o0, o1 = fn(src2, dst2, ast2, adt2, hcat, bias2)
    return jnp.concatenate([o0[:N], o1[:N]], axis=1)


def kernel(x, edge_index, W1, a_src1, a_dst1, b1, ln1_g, ln1_b,
           W2, a_src2, a_dst2, b2, ln2_g, ln2_b, W3, a_src3, a_dst3, b3):
    loop = jnp.arange(N, dtype=edge_index.dtype)
    pad = E2P - E2
    src = jnp.concatenate([edge_index[0], loop, jnp.zeros((pad,), edge_index.dtype)])
    dst = jnp.concatenate([edge_index[1], loop, jnp.full((pad,), TRASH, edge_index.dtype)])
    src = src.reshape(E2P // B, B)
    dst = dst.reshape(E2P // B, B)

    sc_full = _make_sc_layer(HC // 2, C)
    sc_out = _make_sc_layer(NUM_CLASSES // 2, 64)

    as1, ad1 = _attn_mats(a_src1, a_dst1, H, C)
    h0, h1, als, ald = _tc_first(x, W1, as1, ad1)
    acc1 = _sc_edge_stage(sc_full, src, dst, als, ald, h0, h1, b1)

    as2, ad2 = _attn_mats(a_src2, a_dst2, H, C)
    y1, h0, h1, als, ald = _tc_mid(acc1, x, ln1_g, ln1_b, W2, as2, ad2)
    acc2 = _sc_edge_stage(sc_full, src, dst, als, ald, h0, h1, b2)

    as3, ad3 = _attn_mats(a_src3, a_dst3, 1, NUM_CLASSES)
    _, h0, h1, als, ald = _tc_mid(acc2, y1, ln2_g, ln2_b, W3, as3, ad3)
    return _sc_edge_stage(sc_out, src, dst, als, ald, h0, h1, b3)


# speedup vs baseline: 37.1047x; 1.1280x over previous
import functools

import jax
import jax.numpy as jnp
from jax import lax
from jax.experimental import pallas as pl
from jax.experimental.pallas import tpu as pltpu
from jax.experimental.pallas import tpu_sc as plsc

N = 10000
E = 160000
D = 256
H = 8
C = 32
HC = H * C
NUM_CLASSES = 64

NSUB = 16
NP = 10112
TRASH = N
E2 = E + N
B = 64
NS = 3
E2P = 172032
CHUNK = E2P // NSUB
T = CHUNK // B
ROWS = NP // NSUB

BN = 1000


def _tc_first_body(x_ref, w_ref, as_ref, ad_ref, h0_ref, h1_ref, so_ref, do_ref):
    h = jnp.dot(x_ref[...], w_ref[...], preferred_element_type=jnp.float32)
    half = h.shape[1] // 2
    h0_ref[...] = h[:, :half]
    h1_ref[...] = h[:, half:]
    so_ref[...] = jnp.dot(h, as_ref[...], preferred_element_type=jnp.float32)
    do_ref[...] = jnp.dot(h, ad_ref[...], preferred_element_type=jnp.float32)


def _tc_mid_body(z_ref, r_ref, g_ref, b_ref, w_ref, as_ref, ad_ref,
                 y_ref, h0_ref, h1_ref, so_ref, do_ref):
    z = z_ref[...] + r_ref[...]
    mu = jnp.mean(z, axis=-1, keepdims=True)
    var = jnp.mean((z - mu) ** 2, axis=-1, keepdims=True)
    y = (z - mu) / jnp.sqrt(var + 1e-5) * g_ref[...] + b_ref[...]
    y = jnp.where(y > 0, y, jnp.exp(jnp.minimum(y, 0.0)) - 1.0)
    y_ref[...] = y
    h = jnp.dot(y, w_ref[...], preferred_element_type=jnp.float32)
    half = h.shape[1] // 2
    h0_ref[...] = h[:, :half]
    h1_ref[...] = h[:, half:]
    so_ref[...] = jnp.dot(h, as_ref[...], preferred_element_type=jnp.float32)
    do_ref[...] = jnp.dot(h, ad_ref[...], preferred_element_type=jnp.float32)


def _tc_first(x, w, a_s, a_d):
    k, m = w.shape
    half = m // 2
    f32 = jnp.float32
    return pl.pallas_call(
        _tc_first_body,
        grid=(N // BN,),
        in_specs=[
            pl.BlockSpec((BN, k), lambda i: (i, 0)),
            pl.BlockSpec((k, m), lambda i: (0, 0)),
            pl.BlockSpec((m, H), lambda i: (0, 0)),
            pl.BlockSpec((m, H), lambda i: (0, 0)),
        ],
        out_specs=[
            pl.BlockSpec((BN, half), lambda i: (i, 0)),
            pl.BlockSpec((BN, half), lambda i: (i, 0)),
            pl.BlockSpec((BN, H), lambda i: (i, 0)),
            pl.BlockSpec((BN, H), lambda i: (i, 0)),
        ],
        out_shape=[
            jax.ShapeDtypeStruct((N, half), f32),
            jax.ShapeDtypeStruct((N, half), f32),
            jax.ShapeDtypeStruct((N, H), f32),
            jax.ShapeDtypeStruct((N, H), f32),
        ],
    )(x, w, a_s, a_d)


def _tc_mid(z, r, ln_g, ln_b, w, a_s, a_d):
    k, m = w.shape
    half = m // 2
    f32 = jnp.float32
    return pl.pallas_call(
        _tc_mid_body,
        grid=(N // BN,),
        in_specs=[
            pl.BlockSpec((BN, k), lambda i: (i, 0)),
            pl.BlockSpec((BN, k), lambda i: (i, 0)),
            pl.BlockSpec((1, k), lambda i: (0, 0)),
            pl.BlockSpec((1, k), lambda i: (0, 0)),
            pl.BlockSpec((k, m), lambda i: (0, 0)),
            pl.BlockSpec((m, H), lambda i: (0, 0)),
            pl.BlockSpec((m, H), lambda i: (0, 0)),
        ],
        out_specs=[
            pl.BlockSpec((BN, k), lambda i: (i, 0)),
            pl.BlockSpec((BN, half), lambda i: (i, 0)),
            pl.BlockSpec((BN, half), lambda i: (i, 0)),
            pl.BlockSpec((BN, H), lambda i: (i, 0)),
            pl.BlockSpec((BN, H), lambda i: (i, 0)),
        ],
        out_shape=[
            jax.ShapeDtypeStruct((N, k), f32),
            jax.ShapeDtypeStruct((N, half), f32),
            jax.ShapeDtypeStruct((N, half), f32),
            jax.ShapeDtypeStruct((N, H), f32),
            jax.ShapeDtypeStruct((N, H), f32),
        ],
    )(z, r, ln_g.reshape(1, k), ln_b.reshape(1, k), w, a_s, a_d)


def _splat(v, idx):
    dnums = lax.GatherDimensionNumbers(
        offset_dims=(), collapsed_slice_dims=(0,), start_index_map=(0,))
    return lax.gather(v, idx[:, None], dnums, (1,),
                      mode=lax.GatherScatterMode.PROMISE_IN_BOUNDS)


@functools.cache
def _make_sc_layer(half, c_l, interpret=False):
    nj = half // 16
    f32 = jnp.float32
    mesh = plsc.VectorSubcoreMesh(core_axis_name="c", subcore_axis_name="s",
                                  num_cores=2, num_subcores=NSUB)

    def _head_splat(p, c, j):
        if c_l == 32:
            idx = jnp.full((16,), j // 2, jnp.int32) + c * (half // 32)
        else:
            idx = jnp.zeros((16,), jnp.int32)
        return _splat(p, idx)

    def body(*refs):
        (s2_h, d2_h, ast_h, adt_h, hcat_h, bias_h, o0_h, o1_h,
         denom, acc, sbuf, dbuf, sbf2, dsc, asb, adb, pbuf, g, bias_v) = refs[:19]
        sems = refs[19:]
        ISEM = sems[0:NS]
        ASEM = sems[NS:2 * NS]
        BSEM = sems[2 * NS:3 * NS]
        GSEM = sems[3 * NS:4 * NS]
        DSEM = sems[4 * NS:5 * NS]
        SSEM = sems[5 * NS:6 * NS]
        c = lax.axis_index("c")
        s = lax.axis_index("s")
        rbase = s * ROWS
        shift = c * NP

        def idx_issue(t, k):
            row = s * T + t
            pltpu.async_copy(s2_h.at[pl.ds(row, 1)], sbuf.at[pl.ds(k, 1)], ISEM[k])
            pltpu.async_copy(d2_h.at[pl.ds(row, 1)], dbuf.at[pl.ds(k, 1)], ISEM[k])

        def idx_wait(k):
            pltpu.make_async_copy(s2_h.at[pl.ds(0, 1)], sbuf.at[pl.ds(k, 1)], ISEM[k]).wait()
            pltpu.make_async_copy(d2_h.at[pl.ds(0, 1)], dbuf.at[pl.ds(k, 1)], ISEM[k]).wait()

        def gathers_issue(k):
            def sh(q, _):
                sbf2[k, pl.ds(16 * q, 16)] = sbuf[k, pl.ds(16 * q, 16)] + shift
                return 0
            lax.fori_loop(0, B // 16, sh, 0, unroll=True)
            pltpu.async_copy(ast_h.at[sbuf.at[k]], asb.at[k], ASEM[k])
            pltpu.async_copy(adt_h.at[dbuf.at[k]], adb.at[k], BSEM[k])
            pltpu.async_copy(hcat_h.at[sbf2.at[k]], g.at[k], GSEM[k])

        def gathers_wait(k):
            pltpu.make_async_copy(ast_h.at[sbuf.at[k]], asb.at[k], ASEM[k]).wait()
            pltpu.make_async_copy(adt_h.at[dbuf.at[k]], adb.at[k], BSEM[k]).wait()
            pltpu.make_async_copy(hcat_h.at[sbf2.at[k]], g.at[k], GSEM[k]).wait()

        def scatter_issue(k):
            pltpu.async_copy(pbuf.at[k], denom.at[dsc.at[k]], DSEM[k], add=True)
            pltpu.async_copy(g.at[k], acc.at[dsc.at[k]], SSEM[k], add=True)

        def scatter_drain(k):
            pltpu.make_async_copy(pbuf.at[k], denom.at[dsc.at[k]], DSEM[k]).wait()
            pltpu.make_async_copy(g.at[k], acc.at[dsc.at[k]], SSEM[k]).wait()

        pltpu.sync_copy(bias_h.at[pl.ds(c, 1)], bias_v)

        def _fill(i, _):
            for j in range(nj):
                g[0, i, pl.ds(16 * j, 16)] = jnp.zeros((16,), f32)
            pbuf[0, i] = jnp.zeros((16,), f32)
            return 0
        lax.fori_loop(0, B, _fill, 0)
        zchunks = [(q * B, B) for q in range(ROWS // B)]
        if ROWS % B:
            zchunks.append((ROWS - ROWS % B, ROWS % B))
        for off, sz in zchunks:
            pltpu.async_copy(g.at[0, pl.ds(0, sz)], acc.at[pl.ds(rbase + off, sz)], ASEM[0])
            pltpu.async_copy(pbuf.at[0, pl.ds(0, sz)], denom.at[pl.ds(rbase + off, sz)], BSEM[0])
        for off, sz in zchunks:
            pltpu.make_async_copy(g.at[0, pl.ds(0, sz)], acc.at[pl.ds(rbase + off, sz)], ASEM[0]).wait()
            pltpu.make_async_copy(pbuf.at[0, pl.ds(0, sz)], denom.at[pl.ds(rbase + off, sz)], BSEM[0]).wait()
        plsc.subcore_barrier()

        def compute(k):
            def pe(i, _):
                e = asb[k, i] + adb[k, i]
                e = jnp.maximum(e, 0.2 * e)
                pbuf[k, i] = jnp.exp(e)
                return 0
            lax.fori_loop(0, B, pe, 0, unroll=4)

            def me(i, _):
                p = pbuf[k, i]
                for j in range(nj):
                    g[k, i, pl.ds(16 * j, 16)] = g[k, i, pl.ds(16 * j, 16)] * _head_splat(p, c, j)
                return 0
            lax.fori_loop(0, B, me, 0, unroll=2)

        idx_issue(0, 0)
        idx_issue(1, 1)
        idx_wait(0)
        gathers_issue(0)

        def tstep(t3, _):
            for ks in range(NS):
                t = NS * t3 + ks
                k1 = (ks + 1) % NS

                @pl.when(t + 2 < T)
                def _():
                    idx_issue(t + 2, (ks + 2) % NS)

                @pl.when(t + 1 < T)
                def _():
                    idx_wait(k1)

                    @pl.when(t + 1 >= NS)
                    def _():
                        scatter_drain(k1)
                    gathers_issue(k1)
                gathers_wait(ks)
                for q in range(B // 16):
                    dsc[ks, pl.ds(16 * q, 16)] = dbuf[ks, pl.ds(16 * q, 16)]
                compute(ks)
                scatter_issue(ks)
            return 0
        lax.fori_loop(0, T // NS, tstep, 0)
        for ks in range(NS):
            scatter_drain(ks)
        plsc.subcore_barrier()

        rounds = []
        off = 0
        while off < ROWS:
            sizes = []
            for k in range(NS):
                sz = min(B, ROWS - off - sum(sizes))
                if sz > 0:
                    sizes.append(sz)
            rounds.append((off, sizes))
            off += sum(sizes)

        for off, sizes in rounds:
            for k, sz in enumerate(sizes):
                r0 = rbase + off + k * B
                pltpu.async_copy(acc.at[pl.ds(r0, sz)], g.at[k, pl.ds(0, sz)], GSEM[k])
                pltpu.async_copy(denom.at[pl.ds(r0, sz)], asb.at[k, pl.ds(0, sz)], ASEM[k])
            for k, sz in enumerate(sizes):
                r0 = rbase + off + k * B
                pltpu.make_async_copy(acc.at[pl.ds(r0, sz)], g.at[k, pl.ds(0, sz)], GSEM[k]).wait()
                pltpu.make_async_copy(denom.at[pl.ds(r0, sz)], asb.at[k, pl.ds(0, sz)], ASEM[k]).wait()

                def ne(i, _, k=k):
                    d = asb[k, i] + 1e-16
                    for j in range(nj):
                        bv = bias_v[0, pl.ds(16 * j, 16)]
                        g[k, i, pl.ds(16 * j, 16)] = g[k, i, pl.ds(16 * j, 16)] / _head_splat(d, c, j) + bv
                    return 0
                lax.fori_loop(0, sz, ne, 0, unroll=2)

                @pl.when(c == 0)
                def _():
                    pltpu.async_copy(g.at[k, pl.ds(0, sz)], o0_h.at[pl.ds(r0, sz)], BSEM[k])

                @pl.when(c == 1)
                def _():
                    pltpu.async_copy(g.at[k, pl.ds(0, sz)], o1_h.at[pl.ds(r0, sz)], BSEM[k])
            for k, sz in enumerate(sizes):
                r0 = rbase + off + k * B

                @pl.when(c == 0)
                def _():
                    pltpu.make_async_copy(g.at[k, pl.ds(0, sz)], o0_h.at[pl.ds(r0, sz)], BSEM[k]).wait()

                @pl.when(c == 1)
                def _():
                    pltpu.make_async_copy(g.at[k, pl.ds(0, sz)], o1_h.at[pl.ds(r0, sz)], BSEM[k]).wait()

    return pl.kernel(
        body,
        out_type=(
            jax.ShapeDtypeStruct((NP, half), f32),
            jax.ShapeDtypeStruct((NP, half), f32),
        ),
        mesh=mesh,
        interpret=interpret,
        compiler_params=pltpu.CompilerParams(use_tc_tiling_on_sc=False),
        scratch_types=[
            pltpu.VMEM_SHARED((NP, 16), f32),
            pltpu.VMEM_SHARED((NP, half), f32),
            pltpu.VMEM((NS, B), jnp.int32),
            pltpu.VMEM((NS, B), jnp.int32),
            pltpu.VMEM((NS, B), jnp.int32),
            pltpu.VMEM((NS, B), jnp.int32),
            pltpu.VMEM((NS, B, 16), f32),
            pltpu.VMEM((NS, B, 16), f32),
            pltpu.VMEM((NS, B, 16), f32),
            pltpu.VMEM((NS, B, half), f32),
            pltpu.VMEM((1, half), f32),
        ] + [pltpu.SemaphoreType.DMA] * (6 * NS),
    )


def _attn_mats(a_src, a_dst, heads, c_out):
    eye = jnp.eye(heads, dtype=jnp.float32)
    a_s = (a_src[:, :, None] * eye[:, None, :]).reshape(heads * c_out, heads)
    a_d = (a_dst[:, :, None] * eye[:, None, :]).reshape(heads * c_out, heads)
    if heads < H:
        a_s = jnp.pad(a_s, ((0, 0), (0, H - heads)))
        a_d = jnp.pad(a_d, ((0, 0), (0, H - heads)))
    return a_s, a_d


def _pad_tab(a):
    return jnp.pad(a, ((0, NP - N), (0, 16 - H)))


def _sc_edge_stage(fn, src2, dst2, alo, ahi, h0, h1, bias):
    ast = _pad_tab(alo)
    adt = _pad_tab(ahi)
    ast2 = jnp.concatenate([ast, ast], axis=0)
    adt2 = jnp.concatenate([adt, adt], axis=0)
    half = h0.shape[1]
    hcat = jnp.concatenate([jnp.pad(h0, ((0, NP - N), (0, 0))),
                            jnp.pad(h1, ((0, NP - N), (0, 0)))], axis=0)
    bias2 = bias.reshape(2, half)
    o0, o1 = fn(src2, dst2, ast2, adt2, hcat, bias2)
    return jnp.concatenate([o0[:N], o1[:N]], axis=1)


def kernel(x, edge_index, W1, a_src1, a_dst1, b1, ln1_g, ln1_b,
           W2, a_src2, a_dst2, b2, ln2_g, ln2_b, W3, a_src3, a_dst3, b3):
    loop = jnp.arange(N, dtype=edge_index.dtype)
    pad = E2P - E2
    src = jnp.concatenate([edge_index[0], loop, jnp.zeros((pad,), edge_index.dtype)])
    dst = jnp.concatenate([edge_index[1], loop, jnp.full((pad,), TRASH, edge_index.dtype)])
    src = src.reshape(E2P // B, B)
    dst = dst.reshape(E2P // B, B)

    sc_full = _make_sc_layer(HC // 2, C)
    sc_out = _make_sc_layer(NUM_CLASSES // 2, 64)

    as1, ad1 = _attn_mats(a_src1, a_dst1, H, C)
    h0, h1, als, ald = _tc_first(x, W1, as1, ad1)
    acc1 = _sc_edge_stage(sc_full, src, dst, als, ald, h0, h1, b1)

    as2, ad2 = _attn_mats(a_src2, a_dst2, H, C)
    y1, h0, h1, als, ald = _tc_mid(acc1, x, ln1_g, ln1_b, W2, as2, ad2)
    acc2 = _sc_edge_stage(sc_full, src, dst, als, ald, h0, h1, b2)

    as3, ad3 = _attn_mats(a_src3, a_dst3, 1, NUM_CLASSES)
    _, h0, h1, als, ald = _tc_mid(acc2, y1, ln2_g, ln2_b, W3, as3, ad3)
    return _sc_edge_stage(sc_out, src, dst, als, ald, h0, h1, b3)


# speedup vs baseline: 41.5485x; 1.1198x over previous
import functools

import jax
import jax.numpy as jnp
from jax import lax
from jax.experimental import pallas as pl
from jax.experimental.pallas import tpu as pltpu
from jax.experimental.pallas import tpu_sc as plsc

N = 10000
E = 160000
D = 256
H = 8
C = 32
HC = H * C
NUM_CLASSES = 64

NSUB = 16
NP = 10112
TRASH = N
E2 = E + N
B = 64
NS = 3
E2P = 172032
CHUNK = E2P // NSUB
T = CHUNK // B
ROWS = NP // NSUB

BN = 1000


def _tc_first_body(x_ref, w_ref, as_ref, ad_ref, h0_ref, h1_ref, so_ref, do_ref):
    h = jnp.dot(x_ref[...], w_ref[...], preferred_element_type=jnp.float32)
    half = h.shape[1] // 2
    h0_ref[...] = h[:, :half]
    h1_ref[...] = h[:, half:]
    so_ref[...] = jnp.dot(h, as_ref[...], preferred_element_type=jnp.float32)
    do_ref[...] = jnp.dot(h, ad_ref[...], preferred_element_type=jnp.float32)


def _tc_mid_body(z_ref, r_ref, g_ref, b_ref, w_ref, as_ref, ad_ref,
                 y_ref, h0_ref, h1_ref, so_ref, do_ref):
    z = z_ref[...] + r_ref[...]
    mu = jnp.mean(z, axis=-1, keepdims=True)
    var = jnp.mean((z - mu) ** 2, axis=-1, keepdims=True)
    y = (z - mu) / jnp.sqrt(var + 1e-5) * g_ref[...] + b_ref[...]
    y = jnp.where(y > 0, y, jnp.exp(jnp.minimum(y, 0.0)) - 1.0)
    y_ref[...] = y
    h = jnp.dot(y, w_ref[...], preferred_element_type=jnp.float32)
    half = h.shape[1] // 2
    h0_ref[...] = h[:, :half]
    h1_ref[...] = h[:, half:]
    so_ref[...] = jnp.dot(h, as_ref[...], preferred_element_type=jnp.float32)
    do_ref[...] = jnp.dot(h, ad_ref[...], preferred_element_type=jnp.float32)


def _tc_first(x, w, a_s, a_d):
    k, m = w.shape
    half = m // 2
    f32 = jnp.float32
    return pl.pallas_call(
        _tc_first_body,
        grid=(N // BN,),
        in_specs=[
            pl.BlockSpec((BN, k), lambda i: (i, 0)),
            pl.BlockSpec((k, m), lambda i: (0, 0)),
            pl.BlockSpec((m, H), lambda i: (0, 0)),
            pl.BlockSpec((m, H), lambda i: (0, 0)),
        ],
        out_specs=[
            pl.BlockSpec((BN, half), lambda i: (i, 0)),
            pl.BlockSpec((BN, half), lambda i: (i, 0)),
            pl.BlockSpec((BN, H), lambda i: (i, 0)),
            pl.BlockSpec((BN, H), lambda i: (i, 0)),
        ],
        out_shape=[
            jax.ShapeDtypeStruct((N, half), f32),
            jax.ShapeDtypeStruct((N, half), f32),
            jax.ShapeDtypeStruct((N, H), f32),
            jax.ShapeDtypeStruct((N, H), f32),
        ],
    )(x, w, a_s, a_d)


def _tc_mid(z, r, ln_g, ln_b, w, a_s, a_d):
    k, m = w.shape
    half = m // 2
    f32 = jnp.float32
    return pl.pallas_call(
        _tc_mid_body,
        grid=(N // BN,),
        in_specs=[
            pl.BlockSpec((BN, k), lambda i: (i, 0)),
            pl.BlockSpec((BN, k), lambda i: (i, 0)),
            pl.BlockSpec((1, k), lambda i: (0, 0)),
            pl.BlockSpec((1, k), lambda i: (0, 0)),
            pl.BlockSpec((k, m), lambda i: (0, 0)),
            pl.BlockSpec((m, H), lambda i: (0, 0)),
            pl.BlockSpec((m, H), lambda i: (0, 0)),
        ],
        out_specs=[
            pl.BlockSpec((BN, k), lambda i: (i, 0)),
            pl.BlockSpec((BN, half), lambda i: (i, 0)),
            pl.BlockSpec((BN, half), lambda i: (i, 0)),
            pl.BlockSpec((BN, H), lambda i: (i, 0)),
            pl.BlockSpec((BN, H), lambda i: (i, 0)),
        ],
        out_shape=[
            jax.ShapeDtypeStruct((N, k), f32),
            jax.ShapeDtypeStruct((N, half), f32),
            jax.ShapeDtypeStruct((N, half), f32),
            jax.ShapeDtypeStruct((N, H), f32),
            jax.ShapeDtypeStruct((N, H), f32),
        ],
    )(z, r, ln_g.reshape(1, k), ln_b.reshape(1, k), w, a_s, a_d)


def _splat(v, idx):
    dnums = lax.GatherDimensionNumbers(
        offset_dims=(), collapsed_slice_dims=(0,), start_index_map=(0,))
    return lax.gather(v, idx[:, None], dnums, (1,),
                      mode=lax.GatherScatterMode.PROMISE_IN_BOUNDS)


@functools.cache
def _make_sc_layer(half, c_l, interpret=False):
    nj = half // 16
    f32 = jnp.float32
    mesh = plsc.VectorSubcoreMesh(core_axis_name="c", subcore_axis_name="s",
                                  num_cores=2, num_subcores=NSUB)

    def _head_splat(p, c, j):
        if c_l == 32:
            idx = jnp.full((16,), j // 2, jnp.int32) + c * (half // 32)
        else:
            idx = jnp.zeros((16,), jnp.int32)
        return _splat(p, idx)

    WID = half + 16
    TAIL = half

    def body(*refs):
        (sd2_h, adt_h, haug_h, bias_h, o0_h, o1_h,
         acc, sdbuf, sbf2, dsc, adb, g, bias_v) = refs[:13]
        sems = refs[13:]
        ISEM = sems[0:NS]
        BSEM = sems[NS:2 * NS]
        GSEM = sems[2 * NS:3 * NS]
        SSEM = sems[3 * NS:4 * NS]
        c = lax.axis_index("c")
        s = lax.axis_index("s")
        rbase = s * ROWS
        shift = c * NP

        def idx_issue(t, k):
            row = s * T + t
            pltpu.async_copy(sd2_h.at[pl.ds(row, 1)], sdbuf.at[pl.ds(k, 1)], ISEM[k])

        def idx_wait(k):
            pltpu.make_async_copy(sd2_h.at[pl.ds(0, 1)], sdbuf.at[pl.ds(k, 1)], ISEM[k]).wait()

        def gathers_issue(k):
            def sh(q, _):
                sbf2[k, pl.ds(16 * q, 16)] = sdbuf[k, pl.ds(16 * q, 16)] + shift
                dsc[k, pl.ds(16 * q, 16)] = sdbuf[k, pl.ds(B + 16 * q, 16)]
                return 0
            lax.fori_loop(0, B // 16, sh, 0, unroll=True)
            pltpu.async_copy(adt_h.at[dsc.at[k]], adb.at[k], BSEM[k])
            pltpu.async_copy(haug_h.at[sbf2.at[k]], g.at[k], GSEM[k])

        def gathers_wait(k):
            pltpu.make_async_copy(adt_h.at[dsc.at[k]], adb.at[k], BSEM[k]).wait()
            pltpu.make_async_copy(haug_h.at[sbf2.at[k]], g.at[k], GSEM[k]).wait()

        def scatter_issue(k):
            pltpu.async_copy(g.at[k], acc.at[dsc.at[k]], SSEM[k], add=True)

        def scatter_drain(k):
            pltpu.make_async_copy(g.at[k], acc.at[dsc.at[k]], SSEM[k]).wait()

        pltpu.sync_copy(bias_h.at[pl.ds(c, 1)], bias_v)

        def _fill(i, _):
            for j in range(WID // 16):
                g[0, i, pl.ds(16 * j, 16)] = jnp.zeros((16,), f32)
            return 0
        lax.fori_loop(0, B, _fill, 0)
        zchunks = [(q * B, B) for q in range(ROWS // B)]
        if ROWS % B:
            zchunks.append((ROWS - ROWS % B, ROWS % B))
        for off, sz in zchunks:
            pltpu.async_copy(g.at[0, pl.ds(0, sz)], acc.at[pl.ds(rbase + off, sz)], GSEM[0])
        for off, sz in zchunks:
            pltpu.make_async_copy(g.at[0, pl.ds(0, sz)], acc.at[pl.ds(rbase + off, sz)], GSEM[0]).wait()
        plsc.subcore_barrier()

        def compute(k):
            def me(i, _):
                e = g[k, i, pl.ds(TAIL, 16)] + adb[k, i]
                e = jnp.maximum(e, 0.2 * e)
                p = jnp.exp(e)
                g[k, i, pl.ds(TAIL, 16)] = p
                for j in range(nj):
                    g[k, i, pl.ds(16 * j, 16)] = g[k, i, pl.ds(16 * j, 16)] * _head_splat(p, c, j)
                return 0
            lax.fori_loop(0, B, me, 0, unroll=2)

        idx_issue(0, 0)
        idx_issue(1, 1)
        idx_wait(0)
        gathers_issue(0)

        def tstep(t3, _):
            for ks in range(NS):
                t = NS * t3 + ks
                k1 = (ks + 1) % NS

                @pl.when(t + 2 < T)
                def _():
                    idx_issue(t + 2, (ks + 2) % NS)

                @pl.when(t + 1 < T)
                def _():
                    idx_wait(k1)

                    @pl.when(t + 1 >= NS)
                    def _():
                        scatter_drain(k1)
                    gathers_issue(k1)
                gathers_wait(ks)
                compute(ks)
                scatter_issue(ks)
            return 0
        lax.fori_loop(0, T // NS, tstep, 0)
        for ks in range(NS):
            scatter_drain(ks)
        plsc.subcore_barrier()

        rounds = []
        off = 0
        while off < ROWS:
            sizes = []
            for k in range(NS):
                sz = min(B, ROWS - off - sum(sizes))
                if sz > 0:
                    sizes.append(sz)
            rounds.append((off, sizes))
            off += sum(sizes)

        for off, sizes in rounds:
            for k, sz in enumerate(sizes):
                r0 = rbase + off + k * B
                pltpu.async_copy(acc.at[pl.ds(r0, sz)], g.at[k, pl.ds(0, sz)], GSEM[k])
            for k, sz in enumerate(sizes):
                r0 = rbase + off + k * B
                pltpu.make_async_copy(acc.at[pl.ds(r0, sz)], g.at[k, pl.ds(0, sz)], GSEM[k]).wait()

                def ne(i, _, k=k):
                    d = g[k, i, pl.ds(TAIL, 16)] + 1e-16
                    for j in range(nj):
                        bv = bias_v[0, pl.ds(16 * j, 16)]
                        g[k, i, pl.ds(16 * j, 16)] = g[k, i, pl.ds(16 * j, 16)] / _head_splat(d, c, j) + bv
                    return 0
                lax.fori_loop(0, sz, ne, 0, unroll=2)

                @pl.when(c == 0)
                def _():
                    pltpu.async_copy(g.at[k, pl.ds(0, sz)], o0_h.at[pl.ds(r0, sz)], BSEM[k])

                @pl.when(c == 1)
                def _():
                    pltpu.async_copy(g.at[k, pl.ds(0, sz)], o1_h.at[pl.ds(r0, sz)], BSEM[k])
            for k, sz in enumerate(sizes):
                r0 = rbase + off + k * B

                @pl.when(c == 0)
                def _():
                    pltpu.make_async_copy(g.at[k, pl.ds(0, sz)], o0_h.at[pl.ds(r0, sz)], BSEM[k]).wait()

                @pl.when(c == 1)
                def _():
                    pltpu.make_async_copy(g.at[k, pl.ds(0, sz)], o1_h.at[pl.ds(r0, sz)], BSEM[k]).wait()

    return pl.kernel(
        body,
        out_type=(
            jax.ShapeDtypeStruct((NP, WID), f32),
            jax.ShapeDtypeStruct((NP, WID), f32),
        ),
        mesh=mesh,
        interpret=interpret,
        compiler_params=pltpu.CompilerParams(use_tc_tiling_on_sc=False),
        scratch_types=[
            pltpu.VMEM_SHARED((NP, WID), f32),
            pltpu.VMEM((NS, 2 * B), jnp.int32),
            pltpu.VMEM((NS, B), jnp.int32),
            pltpu.VMEM((NS, B), jnp.int32),
            pltpu.VMEM((NS, B, 16), f32),
            pltpu.VMEM((NS, B, WID), f32),
            pltpu.VMEM((1, half), f32),
        ] + [pltpu.SemaphoreType.DMA] * (4 * NS),
    )


def _attn_mats(a_src, a_dst, heads, c_out):
    eye = jnp.eye(heads, dtype=jnp.float32)
    a_s = (a_src[:, :, None] * eye[:, None, :]).reshape(heads * c_out, heads)
    a_d = (a_dst[:, :, None] * eye[:, None, :]).reshape(heads * c_out, heads)
    if heads < H:
        a_s = jnp.pad(a_s, ((0, 0), (0, H - heads)))
        a_d = jnp.pad(a_d, ((0, 0), (0, H - heads)))
    return a_s, a_d


def _pad_tab(a):
    return jnp.pad(a, ((0, NP - N), (0, 16 - H)))


def _sc_edge_stage(fn, sd2, alo, ahi, h0, h1, bias):
    ast = _pad_tab(alo)
    adt = _pad_tab(ahi)
    half = h0.shape[1]
    haug = jnp.concatenate([
        jnp.concatenate([jnp.pad(h0, ((0, NP - N), (0, 0))), ast], axis=1),
        jnp.concatenate([jnp.pad(h1, ((0, NP - N), (0, 0))), ast], axis=1),
    ], axis=0)
    bias2 = bias.reshape(2, half)
    o0, o1 = fn(sd2, adt, haug, bias2)
    return jnp.concatenate([o0[:N, :half], o1[:N, :half]], axis=1)


def kernel(x, edge_index, W1, a_src1, a_dst1, b1, ln1_g, ln1_b,
           W2, a_src2, a_dst2, b2, ln2_g, ln2_b, W3, a_src3, a_dst3, b3):
    loop = jnp.arange(N, dtype=edge_index.dtype)
    pad = E2P - E2
    src = jnp.concatenate([edge_index[0], loop, jnp.zeros((pad,), edge_index.dtype)])
    dst = jnp.concatenate([edge_index[1], loop, jnp.full((pad,), TRASH, edge_index.dtype)])
    sd2 = jnp.concatenate([src.reshape(E2P // B, B), dst.reshape(E2P // B, B)], axis=1)

    sc_full = _make_sc_layer(HC // 2, C)
    sc_out = _make_sc_layer(NUM_CLASSES // 2, 64)

    as1, ad1 = _attn_mats(a_src1, a_dst1, H, C)
    h0, h1, als, ald = _tc_first(x, W1, as1, ad1)
    acc1 = _sc_edge_stage(sc_full, sd2, als, ald, h0, h1, b1)

    as2, ad2 = _attn_mats(a_src2, a_dst2, H, C)
    y1, h0, h1, als, ald = _tc_mid(acc1, x, ln1_g, ln1_b, W2, as2, ad2)
    acc2 = _sc_edge_stage(sc_full, sd2, als, ald, h0, h1, b2)

    as3, ad3 = _attn_mats(a_src3, a_dst3, 1, NUM_CLASSES)
    _, h0, h1, als, ald = _tc_mid(acc2, y1, ln2_g, ln2_b, W3, as3, ad3)
    return _sc_edge_stage(sc_out, sd2, als, ald, h0, h1, b3)


# speedup vs baseline: 45.4906x; 1.0949x over previous
import functools

import jax
import jax.numpy as jnp
from jax import lax
from jax.experimental import pallas as pl
from jax.experimental.pallas import tpu as pltpu
from jax.experimental.pallas import tpu_sc as plsc

N = 10000
E = 160000
D = 256
H = 8
C = 32
HC = H * C
NUM_CLASSES = 64

NSUB = 16
NP = 10112
TRASH = N
E2 = E + N
B = 64
NS = 3
E2P = 172032
CHUNK = E2P // NSUB
T = CHUNK // B
ROWS = NP // NSUB

BN = 1000


def _augment(h, als, half):
    pad = jnp.zeros((h.shape[0], 16 - H), jnp.float32)
    return (jnp.concatenate([h[:, :half], als, pad], axis=1),
            jnp.concatenate([h[:, half:], als, pad], axis=1))


def _tc_first_body(x_ref, w_ref, as_ref, ad_ref, g0_ref, g1_ref, do_ref):
    h = jnp.dot(x_ref[...], w_ref[...], preferred_element_type=jnp.float32)
    half = h.shape[1] // 2
    als = jnp.dot(h, as_ref[...], preferred_element_type=jnp.float32)
    g0_ref[...], g1_ref[...] = _augment(h, als, half)
    do_ref[...] = jnp.dot(h, ad_ref[...], preferred_element_type=jnp.float32)


def _tc_mid_body(z0_ref, z1_ref, r_ref, g_ref, b_ref, w_ref, as_ref, ad_ref,
                 y_ref, g0_ref, g1_ref, do_ref):
    halfp = z0_ref.shape[1] - 16
    z = jnp.concatenate([z0_ref[:, :halfp], z1_ref[:, :halfp]], axis=1) + r_ref[...]
    mu = jnp.mean(z, axis=-1, keepdims=True)
    var = jnp.mean((z - mu) ** 2, axis=-1, keepdims=True)
    y = (z - mu) / jnp.sqrt(var + 1e-5) * g_ref[...] + b_ref[...]
    y = jnp.where(y > 0, y, jnp.exp(jnp.minimum(y, 0.0)) - 1.0)
    y_ref[...] = y
    h = jnp.dot(y, w_ref[...], preferred_element_type=jnp.float32)
    half = h.shape[1] // 2
    als = jnp.dot(h, as_ref[...], preferred_element_type=jnp.float32)
    g0_ref[...], g1_ref[...] = _augment(h, als, half)
    do_ref[...] = jnp.dot(h, ad_ref[...], preferred_element_type=jnp.float32)


def _tc_first(x, w, a_s, a_d):
    k, m = w.shape
    wid = m // 2 + 16
    f32 = jnp.float32
    return pl.pallas_call(
        _tc_first_body,
        grid=(N // BN,),
        in_specs=[
            pl.BlockSpec((BN, k), lambda i: (i, 0)),
            pl.BlockSpec((k, m), lambda i: (0, 0)),
            pl.BlockSpec((m, H), lambda i: (0, 0)),
            pl.BlockSpec((m, H), lambda i: (0, 0)),
        ],
        out_specs=[
            pl.BlockSpec((BN, wid), lambda i: (i, 0)),
            pl.BlockSpec((BN, wid), lambda i: (i, 0)),
            pl.BlockSpec((BN, H), lambda i: (i, 0)),
        ],
        out_shape=[
            jax.ShapeDtypeStruct((N, wid), f32),
            jax.ShapeDtypeStruct((N, wid), f32),
            jax.ShapeDtypeStruct((N, H), f32),
        ],
    )(x, w, a_s, a_d)


def _tc_mid(z0, z1, r, ln_g, ln_b, w, a_s, a_d):
    k, m = w.shape
    widp = z0.shape[1]
    wid = m // 2 + 16
    f32 = jnp.float32
    return pl.pallas_call(
        _tc_mid_body,
        grid=(N // BN,),
        in_specs=[
            pl.BlockSpec((BN, widp), lambda i: (i, 0)),
            pl.BlockSpec((BN, widp), lambda i: (i, 0)),
            pl.BlockSpec((BN, k), lambda i: (i, 0)),
            pl.BlockSpec((1, k), lambda i: (0, 0)),
            pl.BlockSpec((1, k), lambda i: (0, 0)),
            pl.BlockSpec((k, m), lambda i: (0, 0)),
            pl.BlockSpec((m, H), lambda i: (0, 0)),
            pl.BlockSpec((m, H), lambda i: (0, 0)),
        ],
        out_specs=[
            pl.BlockSpec((BN, k), lambda i: (i, 0)),
            pl.BlockSpec((BN, wid), lambda i: (i, 0)),
            pl.BlockSpec((BN, wid), lambda i: (i, 0)),
            pl.BlockSpec((BN, H), lambda i: (i, 0)),
        ],
        out_shape=[
            jax.ShapeDtypeStruct((N, k), f32),
            jax.ShapeDtypeStruct((N, wid), f32),
            jax.ShapeDtypeStruct((N, wid), f32),
            jax.ShapeDtypeStruct((N, H), f32),
        ],
    )(z0, z1, r, ln_g.reshape(1, k), ln_b.reshape(1, k), w, a_s, a_d)


def _splat(v, idx):
    dnums = lax.GatherDimensionNumbers(
        offset_dims=(), collapsed_slice_dims=(0,), start_index_map=(0,))
    return lax.gather(v, idx[:, None], dnums, (1,),
                      mode=lax.GatherScatterMode.PROMISE_IN_BOUNDS)


@functools.cache
def _make_sc_layer(half, c_l, interpret=False):
    nj = half // 16
    f32 = jnp.float32
    mesh = plsc.VectorSubcoreMesh(core_axis_name="c", subcore_axis_name="s",
                                  num_cores=2, num_subcores=NSUB)

    def _head_splat(p, c, j):
        if c_l == 32:
            idx = jnp.full((16,), j // 2, jnp.int32) + c * (half // 32)
        else:
            idx = jnp.zeros((16,), jnp.int32)
        return _splat(p, idx)

    WID = half + 16
    TAIL = half

    def body(*refs):
        (sd2_h, adt_h, h0_h, h1_h, bias_h, o0_h, o1_h,
         acc, sdbuf, dsc, adb, g, bias_v) = refs[:13]
        sems = refs[13:]
        ISEM = sems[0:NS]
        BSEM = sems[NS:2 * NS]
        GSEM = sems[2 * NS:3 * NS]
        SSEM = sems[3 * NS:4 * NS]
        c = lax.axis_index("c")
        s = lax.axis_index("s")
        rbase = s * ROWS

        def idx_issue(t, k):
            row = s * T + t
            pltpu.async_copy(sd2_h.at[pl.ds(row, 1)], sdbuf.at[pl.ds(k, 1)], ISEM[k])

        def idx_wait(k):
            pltpu.make_async_copy(sd2_h.at[pl.ds(0, 1)], sdbuf.at[pl.ds(k, 1)], ISEM[k]).wait()

        def gathers_issue(k):
            def sh(q, _):
                dsc[k, pl.ds(16 * q, 16)] = sdbuf[k, pl.ds(B + 16 * q, 16)]
                return 0
            lax.fori_loop(0, B // 16, sh, 0, unroll=True)
            pltpu.async_copy(adt_h.at[dsc.at[k]], adb.at[k], BSEM[k])

            @pl.when(c == 0)
            def _():
                pltpu.async_copy(h0_h.at[sdbuf.at[k, pl.ds(0, B)]], g.at[k], GSEM[k])

            @pl.when(c == 1)
            def _():
                pltpu.async_copy(h1_h.at[sdbuf.at[k, pl.ds(0, B)]], g.at[k], GSEM[k])

        def gathers_wait(k):
            pltpu.make_async_copy(adt_h.at[dsc.at[k]], adb.at[k], BSEM[k]).wait()
            pltpu.make_async_copy(h0_h.at[sdbuf.at[k, pl.ds(0, B)]], g.at[k], GSEM[k]).wait()

        def scatter_issue(k):
            pltpu.async_copy(g.at[k], acc.at[dsc.at[k]], SSEM[k], add=True)

        def scatter_drain(k):
            pltpu.make_async_copy(g.at[k], acc.at[dsc.at[k]], SSEM[k]).wait()

        pltpu.sync_copy(bias_h.at[pl.ds(c, 1)], bias_v)

        def _fill(i, _):
            for j in range(WID // 16):
                g[0, i, pl.ds(16 * j, 16)] = jnp.zeros((16,), f32)
            return 0
        lax.fori_loop(0, B, _fill, 0)
        zchunks = [(q * B, B) for q in range(ROWS // B)]
        if ROWS % B:
            zchunks.append((ROWS - ROWS % B, ROWS % B))
        for off, sz in zchunks:
            pltpu.async_copy(g.at[0, pl.ds(0, sz)], acc.at[pl.ds(rbase + off, sz)], GSEM[0])
        for off, sz in zchunks:
            pltpu.make_async_copy(g.at[0, pl.ds(0, sz)], acc.at[pl.ds(rbase + off, sz)], GSEM[0]).wait()
        plsc.subcore_barrier()

        def compute(k):
            def me(i, _):
                e = g[k, i, pl.ds(TAIL, 16)] + adb[k, i]
                e = jnp.maximum(e, 0.2 * e)
                p = jnp.exp(e)
                g[k, i, pl.ds(TAIL, 16)] = p
                for j in range(nj):
                    g[k, i, pl.ds(16 * j, 16)] = g[k, i, pl.ds(16 * j, 16)] * _head_splat(p, c, j)
                return 0
            lax.fori_loop(0, B, me, 0, unroll=2)

        idx_issue(0, 0)
        idx_issue(1, 1)
        idx_wait(0)
        gathers_issue(0)

        def tstep(t3, _):
            for ks in range(NS):
                t = NS * t3 + ks
                k1 = (ks + 1) % NS

                @pl.when(t + 2 < T)
                def _():
                    idx_issue(t + 2, (ks + 2) % NS)

                @pl.when(t + 1 < T)
                def _():
                    idx_wait(k1)

                    @pl.when(t + 1 >= NS)
                    def _():
                        scatter_drain(k1)
                    gathers_issue(k1)
                gathers_wait(ks)
                compute(ks)
                scatter_issue(ks)
            return 0
        lax.fori_loop(0, T // NS, tstep, 0)
        for ks in range(NS):
            scatter_drain(ks)
        plsc.subcore_barrier()

        rounds = []
        off = 0
        while off < ROWS:
            sizes = []
            for k in range(NS):
                sz = min(B, ROWS - off - sum(sizes))
                if sz > 0:
                    sizes.append(sz)
            rounds.append((off, sizes))
            off += sum(sizes)

        for off, sizes in rounds:
            for k, sz in enumerate(sizes):
                r0 = rbase + off + k * B
                pltpu.async_copy(acc.at[pl.ds(r0, sz)], g.at[k, pl.ds(0, sz)], GSEM[k])
            for k, sz in enumerate(sizes):
                r0 = rbase + off + k * B
                pltpu.make_async_copy(acc.at[pl.ds(r0, sz)], g.at[k, pl.ds(0, sz)], GSEM[k]).wait()

                def ne(i, _, k=k):
                    d = g[k, i, pl.ds(TAIL, 16)] + 1e-16
                    for j in range(nj):
                        bv = bias_v[0, pl.ds(16 * j, 16)]
                        g[k, i, pl.ds(16 * j, 16)] = g[k, i, pl.ds(16 * j, 16)] / _head_splat(d, c, j) + bv
                    return 0
                lax.fori_loop(0, sz, ne, 0, unroll=2)

                @pl.when(c == 0)
                def _():
                    pltpu.async_copy(g.at[k, pl.ds(0, sz)], o0_h.at[pl.ds(r0, sz)], BSEM[k])

                @pl.when(c == 1)
                def _():
                    pltpu.async_copy(g.at[k, pl.ds(0, sz)], o1_h.at[pl.ds(r0, sz)], BSEM[k])
            for k, sz in enumerate(sizes):
                r0 = rbase + off + k * B

                @pl.when(c == 0)
                def _():
                    pltpu.make_async_copy(g.at[k, pl.ds(0, sz)], o0_h.at[pl.ds(r0, sz)], BSEM[k]).wait()

                @pl.when(c == 1)
                def _():
                    pltpu.make_async_copy(g.at[k, pl.ds(0, sz)], o1_h.at[pl.ds(r0, sz)], BSEM[k]).wait()

    return pl.kernel(
        body,
        out_type=(
            jax.ShapeDtypeStruct((NP, WID), f32),
            jax.ShapeDtypeStruct((NP, WID), f32),
        ),
        mesh=mesh,
        interpret=interpret,
        compiler_params=pltpu.CompilerParams(use_tc_tiling_on_sc=False),
        scratch_types=[
            pltpu.VMEM_SHARED((NP, WID), f32),
            pltpu.VMEM((NS, 2 * B), jnp.int32),
            pltpu.VMEM((NS, B), jnp.int32),
            pltpu.VMEM((NS, B, 16), f32),
            pltpu.VMEM((NS, B, WID), f32),
            pltpu.VMEM((1, half), f32),
        ] + [pltpu.SemaphoreType.DMA] * (4 * NS),
    )


def _attn_mats(a_src, a_dst, heads, c_out):
    eye = jnp.eye(heads, dtype=jnp.float32)
    a_s = (a_src[:, :, None] * eye[:, None, :]).reshape(heads * c_out, heads)
    a_d = (a_dst[:, :, None] * eye[:, None, :]).reshape(heads * c_out, heads)
    if heads < H:
        a_s = jnp.pad(a_s, ((0, 0), (0, H - heads)))
        a_d = jnp.pad(a_d, ((0, 0), (0, H - heads)))
    return a_s, a_d


def _pad_tab(a):
    return jnp.pad(a, ((0, NP - N), (0, 16 - H)))


def _sc_edge_stage(fn, sd2, ald, g0, g1, bias):
    half = g0.shape[1] - 16
    return fn(sd2, _pad_tab(ald), g0, g1, bias.reshape(2, half))


def kernel(x, edge_index, W1, a_src1, a_dst1, b1, ln1_g, ln1_b,
           W2, a_src2, a_dst2, b2, ln2_g, ln2_b, W3, a_src3, a_dst3, b3):
    loop = jnp.arange(N, dtype=edge_index.dtype)
    pad = E2P - E2
    src = jnp.concatenate([edge_index[0], loop, jnp.zeros((pad,), edge_index.dtype)])
    dst = jnp.concatenate([edge_index[1], loop, jnp.full((pad,), TRASH, edge_index.dtype)])
    sd2 = jnp.concatenate([src.reshape(E2P // B, B), dst.reshape(E2P // B, B)], axis=1)

    sc_full = _make_sc_layer(HC // 2, C)
    sc_out = _make_sc_layer(NUM_CLASSES // 2, 64)

    as1, ad1 = _attn_mats(a_src1, a_dst1, H, C)
    g0, g1, ald = _tc_first(x, W1, as1, ad1)
    o0, o1 = _sc_edge_stage(sc_full, sd2, ald, g0, g1, b1)

    as2, ad2 = _attn_mats(a_src2, a_dst2, H, C)
    y1, g0, g1, ald = _tc_mid(o0, o1, x, ln1_g, ln1_b, W2, as2, ad2)
    o0, o1 = _sc_edge_stage(sc_full, sd2, ald, g0, g1, b2)

    as3, ad3 = _attn_mats(a_src3, a_dst3, 1, NUM_CLASSES)
    _, g0, g1, ald = _tc_mid(o0, o1, y1, ln2_g, ln2_b, W3, as3, ad3)
    o0, o1 = _sc_edge_stage(sc_out, sd2, ald, g0, g1, b3)
    half = NUM_CLASSES // 2
    return jnp.concatenate([o0[:N, :half], o1[:N, :half]], axis=1)
